# Initial kernel scaffold; baseline (speedup 1.0000x reference)
import jax
import jax.numpy as jnp
from jax.experimental import pallas as pl


def kernel(vecs, c_sum, c_count):
    raise NotImplementedError("write your pallas kernel here")


# speedup vs baseline: 3.8866x; 3.8866x over previous
import jax
import jax.numpy as jnp
from jax import lax
from jax.experimental import pallas as pl
from jax.experimental.pallas import tpu as pltpu

_B, _H, _R, _C, _K, _S = 2, 8, 16, 128, 64, 512
_RC = _R * _C
_BN = 512
_NB = _RC // _BN
_GRID = _B * _H * _NB


def _vq_body(vecs_ref, csum_ref, ccnt_ref, vq_ref, z_ref, e_ref, commit_ref):
    v = vecs_ref[0, 0]
    c = csum_ref[0] / jnp.maximum(ccnt_ref[0], 0.01)
    dot = lax.dot_general(v, c, (((1,), (1,)), ((), ())),
                          preferred_element_type=jnp.float32)
    vnorm = jnp.sum(v * v, axis=1, keepdims=True)
    cnorm = jnp.sum(c * c, axis=1)
    d2 = (vnorm - 2.0 * dot) + cnorm[None, :]
    mind = jnp.min(d2, axis=1, keepdims=True)
    iota = lax.broadcasted_iota(jnp.int32, (_BN, _S), 1)
    z = jnp.min(jnp.where(d2 == mind, iota, _S), axis=1, keepdims=True)
    onehot = (iota == z).astype(jnp.float32)
    cz = lax.dot_general(onehot, c, (((1,), (0,)), ((), ())),
                         preferred_element_type=jnp.float32)
    vq_ref[0, 0] = cz
    z_ref[0] = z
    e = jnp.maximum(mind, 0.0)
    e_ref[0] = e
    i = pl.program_id(0)
    prev = jnp.where(i == 0, 0.0, commit_ref[0, 0])
    commit_ref[0, 0] = prev + jnp.sum(e)


def kernel(vecs, c_sum, c_count):
    v4 = vecs.reshape(_B, _H, _RC, _K)
    ccnt = c_count.reshape(_H, _S, 1)

    def im_v(i):
        return (i // (_H * _NB), (i // _NB) % _H, i % _NB, 0)

    def im_cb(i):
        return ((i // _NB) % _H, 0, 0)

    vq, z_out, e_out, commit = pl.pallas_call(
        _vq_body,
        grid=(_GRID,),
        in_specs=[
            pl.BlockSpec((1, 1, _BN, _K), im_v),
            pl.BlockSpec((1, _S, _K), im_cb),
            pl.BlockSpec((1, _S, 1), im_cb),
        ],
        out_specs=[
            pl.BlockSpec((1, 1, _BN, _K), im_v),
            pl.BlockSpec((1, _BN, 1), lambda i: (i, 0, 0)),
            pl.BlockSpec((1, _BN, 1), lambda i: (i, 0, 0)),
            pl.BlockSpec((1, 1), lambda i: (0, 0), memory_space=pltpu.SMEM),
        ],
        out_shape=[
            jax.ShapeDtypeStruct((_B, _H, _RC, _K), jnp.float32),
            jax.ShapeDtypeStruct((_GRID, _BN, 1), jnp.int32),
            jax.ShapeDtypeStruct((_GRID, _BN, 1), jnp.float32),
            jax.ShapeDtypeStruct((1, 1), jnp.float32),
        ],
    )(v4, c_sum, ccnt)

    vecs_hat = vq.reshape(_B, _H, _R, _C, _K)
    z = z_out.reshape(_B, _H, _R, _C)
    errs2 = e_out.reshape(_B, _H, _R, _C)
    l_commit = commit[0, 0] / jnp.float32(_B * _R * _C)
    l_codebook = jnp.zeros((), jnp.float32)
    return (vecs_hat, z, l_commit, l_codebook, errs2)


# speedup vs baseline: 4.5846x; 1.1796x over previous
import jax
import jax.numpy as jnp
from jax import lax
from jax.experimental import pallas as pl
from jax.experimental.pallas import tpu as pltpu

_B, _H, _R, _C, _K, _S = 2, 8, 16, 128, 64, 512
_RC = _R * _C
_BN = 1024
_NB = _RC // _BN
_J = _B * _NB


def _vq_body(vecs_ref, csum_ref, ccnt_ref, vq_ref, z_ref, e_ref, commit_ref,
             c_s, cn_s):
    h = pl.program_id(0)
    j = pl.program_id(1)

    @pl.when(j == 0)
    def _():
        c0 = csum_ref[0] / jnp.maximum(ccnt_ref[0], 0.01)
        c_s[...] = c0
        cn_s[...] = jnp.sum(c0 * c0, axis=1)[None, :]

    v = vecs_ref[0, 0, 0]
    c = c_s[...]
    dot = lax.dot_general(v, c, (((1,), (1,)), ((), ())),
                          preferred_element_type=jnp.float32)
    vnorm = jnp.sum(v * v, axis=1, keepdims=True)
    d2 = (vnorm - 2.0 * dot) + cn_s[...]
    mind = jnp.min(d2, axis=1, keepdims=True)
    iota = lax.broadcasted_iota(jnp.int32, (_BN, _S), 1)
    z = jnp.min(jnp.where(d2 == mind, iota, _S), axis=1, keepdims=True)
    onehot = (iota == z).astype(jnp.float32)
    cz = lax.dot_general(onehot, c, (((1,), (0,)), ((), ())),
                         preferred_element_type=jnp.float32)
    vq_ref[0, 0, 0] = cz
    z_ref[0, 0, 0] = z
    e = jnp.maximum(mind, 0.0)
    e_ref[0, 0, 0] = e
    prev = jnp.where((h == 0) & (j == 0), 0.0, commit_ref[0, 0])
    commit_ref[0, 0] = prev + jnp.sum(e)


def kernel(vecs, c_sum, c_count):
    v5 = vecs.reshape(_B, _H, _NB, _BN, _K)
    ccnt = c_count.reshape(_H, _S, 1)

    def im_v(h, j):
        return (j // _NB, h, j % _NB, 0, 0)

    def im_cb(h, j):
        return (h, 0, 0)

    vq, z_out, e_out, commit = pl.pallas_call(
        _vq_body,
        grid=(_H, _J),
        in_specs=[
            pl.BlockSpec((1, 1, 1, _BN, _K), im_v),
            pl.BlockSpec((1, _S, _K), im_cb),
            pl.BlockSpec((1, _S, 1), im_cb),
        ],
        out_specs=[
            pl.BlockSpec((1, 1, 1, _BN, _K), im_v),
            pl.BlockSpec((1, 1, 1, _BN, 1), im_v),
            pl.BlockSpec((1, 1, 1, _BN, 1), im_v),
            pl.BlockSpec((1, 1), lambda h, j: (0, 0), memory_space=pltpu.SMEM),
        ],
        out_shape=[
            jax.ShapeDtypeStruct((_B, _H, _NB, _BN, _K), jnp.float32),
            jax.ShapeDtypeStruct((_B, _H, _NB, _BN, 1), jnp.int32),
            jax.ShapeDtypeStruct((_B, _H, _NB, _BN, 1), jnp.float32),
            jax.ShapeDtypeStruct((1, 1), jnp.float32),
        ],
        scratch_shapes=[
            pltpu.VMEM((_S, _K), jnp.float32),
            pltpu.VMEM((1, _S), jnp.float32),
        ],
    )(v5, c_sum, ccnt)

    vecs_hat = vq.reshape(_B, _H, _R, _C, _K)
    z = z_out.reshape(_B, _H, _R, _C)
    errs2 = e_out.reshape(_B, _H, _R, _C)
    l_commit = commit[0, 0] / jnp.float32(_B * _R * _C)
    l_codebook = jnp.zeros((), jnp.float32)
    return (vecs_hat, z, l_commit, l_codebook, errs2)


# speedup vs baseline: 5.0861x; 1.1094x over previous
import jax
import jax.numpy as jnp
from jax import lax
from jax.experimental import pallas as pl
from jax.experimental.pallas import tpu as pltpu

_B, _H, _R, _C, _K, _S = 2, 8, 16, 128, 64, 512
_RC = _R * _C
_BN = 1024
_NB = _RC // _BN
_J = _B * _NB


def _vq_body(vecs_ref, csum_ref, ccnt_ref, vq_ref, z_ref, e_ref, commit_ref,
             c_s, cn_s):
    h = pl.program_id(0)
    j = pl.program_id(1)

    @pl.when(j == 0)
    def _():
        c0 = csum_ref[0] / jnp.maximum(ccnt_ref[0], 0.01)
        c_s[...] = c0
        cn_s[...] = jnp.sum(c0 * c0, axis=1)[None, :]

    v = vecs_ref[0, 0, 0]
    c = c_s[...]
    dot = lax.dot_general(v, c, (((1,), (1,)), ((), ())),
                          preferred_element_type=jnp.float32)
    vnorm = jnp.sum(v * v, axis=1, keepdims=True)
    d2 = (vnorm - 2.0 * dot) + cn_s[...]
    mind = jnp.min(d2, axis=1, keepdims=True)
    iota = lax.broadcasted_iota(jnp.int32, (_BN, _S), 1)
    z = jnp.min(jnp.where(d2 == mind, iota, _S), axis=1, keepdims=True)
    onehot = (iota == z).astype(jnp.float32)
    cz = lax.dot_general(onehot, c, (((1,), (0,)), ((), ())),
                         preferred_element_type=jnp.float32)
    vq_ref[0, 0, 0] = cz
    z_ref[0, 0, 0] = z.reshape(_BN // 128, 128)
    e = jnp.maximum(mind, 0.0)
    e_ref[0, 0, 0] = e.reshape(_BN // 128, 128)
    prev = jnp.where((h == 0) & (j == 0), 0.0, commit_ref[0, 0])
    commit_ref[0, 0] = prev + jnp.sum(e)


def kernel(vecs, c_sum, c_count):
    v5 = vecs.reshape(_B, _H, _NB, _BN, _K)
    ccnt = c_count.reshape(_H, _S, 1)

    def im_v(h, j):
        return (j // _NB, h, j % _NB, 0, 0)

    def im_cb(h, j):
        return (h, 0, 0)

    vq, z_out, e_out, commit = pl.pallas_call(
        _vq_body,
        grid=(_H, _J),
        in_specs=[
            pl.BlockSpec((1, 1, 1, _BN, _K), im_v),
            pl.BlockSpec((1, _S, _K), im_cb),
            pl.BlockSpec((1, _S, 1), im_cb),
        ],
        out_specs=[
            pl.BlockSpec((1, 1, 1, _BN, _K), im_v),
            pl.BlockSpec((1, 1, 1, _BN // 128, 128), im_v),
            pl.BlockSpec((1, 1, 1, _BN // 128, 128), im_v),
            pl.BlockSpec((1, 1), lambda h, j: (0, 0), memory_space=pltpu.SMEM),
        ],
        out_shape=[
            jax.ShapeDtypeStruct((_B, _H, _NB, _BN, _K), jnp.float32),
            jax.ShapeDtypeStruct((_B, _H, _NB, _BN // 128, 128), jnp.int32),
            jax.ShapeDtypeStruct((_B, _H, _NB, _BN // 128, 128), jnp.float32),
            jax.ShapeDtypeStruct((1, 1), jnp.float32),
        ],
        scratch_shapes=[
            pltpu.VMEM((_S, _K), jnp.float32),
            pltpu.VMEM((1, _S), jnp.float32),
        ],
    )(v5, c_sum, ccnt)

    vecs_hat = vq.reshape(_B, _H, _R, _C, _K)
    z = z_out.reshape(_B, _H, _R, _C)
    errs2 = e_out.reshape(_B, _H, _R, _C)
    l_commit = commit[0, 0] / jnp.float32(_B * _R * _C)
    l_codebook = jnp.zeros((), jnp.float32)
    return (vecs_hat, z, l_commit, l_codebook, errs2)


# speedup vs baseline: 5.6742x; 1.1156x over previous
import jax
import jax.numpy as jnp
from jax import lax
from jax.experimental import pallas as pl
from jax.experimental.pallas import tpu as pltpu

_B, _H, _R, _C, _K, _S = 2, 8, 16, 128, 64, 512


def _vq_body(vecs_ref, csum_ref, ccnt_ref, vq_ref, z_ref, e_ref, commit_ref,
             c_s, cn_s):
    h = pl.program_id(0)
    b = pl.program_id(1)

    @pl.when(b == 0)
    def _():
        c0 = csum_ref[0] / jnp.maximum(ccnt_ref[0], 0.01)
        c_s[...] = c0
        cn_s[...] = jnp.sum(c0 * c0, axis=1, keepdims=True)

    c = c_s[...]
    cn = cn_s[...]
    iota0 = lax.broadcasted_iota(jnp.int32, (_S, _C), 0)
    esum = None
    for r in range(_R):
        v = vecs_ref[0, 0, r]
        dot = lax.dot_general(c, v, (((1,), (0,)), ((), ())),
                              preferred_element_type=jnp.float32)
        vnorm = jnp.sum(v * v, axis=0, keepdims=True)
        d2 = (vnorm - 2.0 * dot) + cn
        mind = jnp.min(d2, axis=0, keepdims=True)
        z = jnp.min(jnp.where(d2 == mind, iota0, _S), axis=0, keepdims=True)
        onehot = (iota0 == z).astype(jnp.float32)
        cz = lax.dot_general(c, onehot, (((0,), (0,)), ((), ())),
                             preferred_element_type=jnp.float32)
        vq_ref[0, 0, r] = cz
        z_ref[0, 0, r] = z[0]
        e = jnp.maximum(mind, 0.0)
        e_ref[0, 0, r] = e[0]
        s = jnp.sum(e)
        esum = s if esum is None else esum + s
    prev = jnp.where((h == 0) & (b == 0), 0.0, commit_ref[0, 0])
    commit_ref[0, 0] = prev + esum


def kernel(vecs, c_sum, c_count):
    vt = jnp.swapaxes(vecs, 3, 4)
    ccnt = c_count.reshape(_H, _S, 1)

    def im_v(h, b):
        return (b, h, 0, 0, 0)

    def im_cb(h, b):
        return (h, 0, 0)

    def im_ze(h, b):
        return (b, h, 0, 0)

    vq, z_out, e_out, commit = pl.pallas_call(
        _vq_body,
        grid=(_H, _B),
        in_specs=[
            pl.BlockSpec((1, 1, _R, _K, _C), im_v),
            pl.BlockSpec((1, _S, _K), im_cb),
            pl.BlockSpec((1, _S, 1), im_cb),
        ],
        out_specs=[
            pl.BlockSpec((1, 1, _R, _K, _C), im_v),
            pl.BlockSpec((1, 1, _R, _C), im_ze),
            pl.BlockSpec((1, 1, _R, _C), im_ze),
            pl.BlockSpec((1, 1), lambda h, b: (0, 0), memory_space=pltpu.SMEM),
        ],
        out_shape=[
            jax.ShapeDtypeStruct((_B, _H, _R, _K, _C), jnp.float32),
            jax.ShapeDtypeStruct((_B, _H, _R, _C), jnp.int32),
            jax.ShapeDtypeStruct((_B, _H, _R, _C), jnp.float32),
            jax.ShapeDtypeStruct((1, 1), jnp.float32),
        ],
        scratch_shapes=[
            pltpu.VMEM((_S, _K), jnp.float32),
            pltpu.VMEM((_S, 1), jnp.float32),
        ],
    )(vt, c_sum, ccnt)

    vecs_hat = jnp.swapaxes(vq, 3, 4)
    l_commit = commit[0, 0] / jnp.float32(_B * _R * _C)
    l_codebook = jnp.zeros((), jnp.float32)
    return (vecs_hat, z_out, l_commit, l_codebook, e_out)


# speedup vs baseline: 11.8433x; 2.0872x over previous
import jax
import jax.numpy as jnp
from jax import lax
from jax.experimental import pallas as pl
from jax.experimental.pallas import tpu as pltpu

_B, _H, _R, _C, _K, _S = 2, 8, 16, 128, 64, 512
_N = _R * _C


def _vq_body(vecs_ref, csum_ref, ccnt_ref, vq_ref, z_ref, e_ref, commit_ref,
             c_s, cn_s):
    h = pl.program_id(0)
    b = pl.program_id(1)

    @pl.when(b == 0)
    def _():
        c0 = csum_ref[0] / jnp.maximum(ccnt_ref[0], 0.01)
        c_s[...] = c0
        cn_s[...] = jnp.sum(c0 * c0, axis=1, keepdims=True)

    c = c_s[...]
    cn = cn_s[...]
    v = jnp.concatenate([vecs_ref[0, 0, r] for r in range(_R)], axis=1)
    dot = lax.dot_general(c, v, (((1,), (0,)), ((), ())),
                          preferred_element_type=jnp.float32)
    vnorm = jnp.sum(v * v, axis=0, keepdims=True)
    d2 = (vnorm - 2.0 * dot) + cn
    mind = jnp.min(d2, axis=0, keepdims=True)
    iota0 = lax.broadcasted_iota(jnp.int32, (_S, _N), 0)
    z = jnp.min(jnp.where(d2 == mind, iota0, _S), axis=0, keepdims=True)
    onehot = (iota0 == z).astype(jnp.float32)
    cz = lax.dot_general(c, onehot, (((0,), (0,)), ((), ())),
                         preferred_element_type=jnp.float32)
    for r in range(_R):
        vq_ref[0, 0, r] = cz[:, r * _C:(r + 1) * _C]
    z_ref[0, 0] = z.reshape(_R, _C)
    e = jnp.maximum(mind, 0.0)
    e_ref[0, 0] = e.reshape(_R, _C)
    prev = jnp.where((h == 0) & (b == 0), 0.0, commit_ref[0, 0])
    commit_ref[0, 0] = prev + jnp.sum(e)


def kernel(vecs, c_sum, c_count):
    vt = jnp.swapaxes(vecs, 3, 4)
    ccnt = c_count.reshape(_H, _S, 1)

    def im_v(h, b):
        return (b, h, 0, 0, 0)

    def im_cb(h, b):
        return (h, 0, 0)

    def im_ze(h, b):
        return (b, h, 0, 0)

    vq, z_out, e_out, commit = pl.pallas_call(
        _vq_body,
        grid=(_H, _B),
        in_specs=[
            pl.BlockSpec((1, 1, _R, _K, _C), im_v),
            pl.BlockSpec((1, _S, _K), im_cb),
            pl.BlockSpec((1, _S, 1), im_cb),
        ],
        out_specs=[
            pl.BlockSpec((1, 1, _R, _K, _C), im_v),
            pl.BlockSpec((1, 1, _R, _C), im_ze),
            pl.BlockSpec((1, 1, _R, _C), im_ze),
            pl.BlockSpec((1, 1), lambda h, b: (0, 0), memory_space=pltpu.SMEM),
        ],
        out_shape=[
            jax.ShapeDtypeStruct((_B, _H, _R, _K, _C), jnp.float32),
            jax.ShapeDtypeStruct((_B, _H, _R, _C), jnp.int32),
            jax.ShapeDtypeStruct((_B, _H, _R, _C), jnp.float32),
            jax.ShapeDtypeStruct((1, 1), jnp.float32),
        ],
        scratch_shapes=[
            pltpu.VMEM((_S, _K), jnp.float32),
            pltpu.VMEM((_S, 1), jnp.float32),
        ],
    )(vt, c_sum, ccnt)

    vecs_hat = jnp.swapaxes(vq, 3, 4)
    l_commit = commit[0, 0] / jnp.float32(_B * _R * _C)
    l_codebook = jnp.zeros((), jnp.float32)
    return (vecs_hat, z_out, l_commit, l_codebook, e_out)


# speedup vs baseline: 12.2491x; 1.0343x over previous
import jax
import jax.numpy as jnp
from jax import lax
from jax.experimental import pallas as pl
from jax.experimental.pallas import tpu as pltpu

_B, _H, _R, _C, _K, _S = 2, 8, 16, 128, 64, 512
_N = _R * _C


def _vq_body(vecs_ref, csum_ref, ccnt_ref, vq_ref, z_ref, e_ref, commit_ref,
             c_s, c2_s, cn_s):
    h = pl.program_id(0)
    b = pl.program_id(1)

    @pl.when(b == 0)
    def _():
        c0 = csum_ref[0] / jnp.maximum(ccnt_ref[0], 0.01)
        c_s[...] = c0
        c2_s[...] = -2.0 * c0
        cn_s[...] = jnp.sum(c0 * c0, axis=1, keepdims=True)

    c = c_s[...]
    cn = cn_s[...]
    v = jnp.concatenate([vecs_ref[0, 0, r] for r in range(_R)], axis=1)
    dot2 = lax.dot_general(c2_s[...], v, (((1,), (0,)), ((), ())),
                           preferred_element_type=jnp.float32)
    vnorm = jnp.sum(v * v, axis=0, keepdims=True)
    d2 = (vnorm + dot2) + cn
    mind = jnp.min(d2, axis=0, keepdims=True)
    iota0 = lax.broadcasted_iota(jnp.int32, (_S, _N), 0)
    z = jnp.min(jnp.where(d2 == mind, iota0, _S), axis=0, keepdims=True)
    onehot = (iota0 == z).astype(jnp.float32)
    cz = lax.dot_general(c, onehot, (((0,), (0,)), ((), ())),
                         preferred_element_type=jnp.float32)
    for r in range(_R):
        vq_ref[0, 0, r] = cz[:, r * _C:(r + 1) * _C]
    z_ref[0, 0] = z.reshape(_R, _C)
    e = jnp.maximum(mind, 0.0)
    e_ref[0, 0] = e.reshape(_R, _C)
    prev = jnp.where((h == 0) & (b == 0), 0.0, commit_ref[0, 0])
    commit_ref[0, 0] = prev + jnp.sum(e)


def kernel(vecs, c_sum, c_count):
    vt = jnp.swapaxes(vecs, 3, 4)
    ccnt = c_count.reshape(_H, _S, 1)

    def im_v(h, b):
        return (b, h, 0, 0, 0)

    def im_cb(h, b):
        return (h, 0, 0)

    def im_ze(h, b):
        return (b, h, 0, 0)

    vq, z_out, e_out, commit = pl.pallas_call(
        _vq_body,
        grid=(_H, _B),
        in_specs=[
            pl.BlockSpec((1, 1, _R, _K, _C), im_v),
            pl.BlockSpec((1, _S, _K), im_cb),
            pl.BlockSpec((1, _S, 1), im_cb),
        ],
        out_specs=[
            pl.BlockSpec((1, 1, _R, _K, _C), im_v),
            pl.BlockSpec((1, 1, _R, _C), im_ze),
            pl.BlockSpec((1, 1, _R, _C), im_ze),
            pl.BlockSpec((1, 1), lambda h, b: (0, 0), memory_space=pltpu.SMEM),
        ],
        out_shape=[
            jax.ShapeDtypeStruct((_B, _H, _R, _K, _C), jnp.float32),
            jax.ShapeDtypeStruct((_B, _H, _R, _C), jnp.int32),
            jax.ShapeDtypeStruct((_B, _H, _R, _C), jnp.float32),
            jax.ShapeDtypeStruct((1, 1), jnp.float32),
        ],
        scratch_shapes=[
            pltpu.VMEM((_S, _K), jnp.float32),
            pltpu.VMEM((_S, _K), jnp.float32),
            pltpu.VMEM((_S, 1), jnp.float32),
        ],
    )(vt, c_sum, ccnt)

    vecs_hat = jnp.swapaxes(vq, 3, 4)
    l_commit = commit[0, 0] / jnp.float32(_B * _R * _C)
    l_codebook = jnp.zeros((), jnp.float32)
    return (vecs_hat, z_out, l_commit, l_codebook, e_out)


# speedup vs baseline: 12.4544x; 1.0168x over previous
import jax
import jax.numpy as jnp
from jax import lax
from jax.experimental import pallas as pl
from jax.experimental.pallas import tpu as pltpu

_B, _H, _R, _C, _K, _S = 2, 8, 16, 128, 64, 512
_N = _R * _C


def _vq_body(vecs_ref, csum_ref, ccnt_ref, vq_ref, z_ref, e_ref, commit_ref,
             c_s, c2_s, cn_s, iota_s):
    h = pl.program_id(0)
    b = pl.program_id(1)

    @pl.when((h == 0) & (b == 0))
    def _():
        iota_s[...] = lax.broadcasted_iota(jnp.int32, (_S, _N), 0).astype(
            jnp.float32)

    @pl.when(b == 0)
    def _():
        c0 = csum_ref[0] / jnp.maximum(ccnt_ref[0], 0.01)
        c_s[...] = c0
        c2_s[...] = -2.0 * c0
        cn_s[...] = jnp.sum(c0 * c0, axis=1, keepdims=True)

    c = c_s[...]
    cn = cn_s[...]
    v = jnp.concatenate([vecs_ref[0, 0, r] for r in range(_R)], axis=1)
    dot2 = lax.dot_general(c2_s[...], v, (((1,), (0,)), ((), ())),
                           preferred_element_type=jnp.float32)
    vnorm = jnp.sum(v * v, axis=0, keepdims=True)
    d2 = (vnorm + dot2) + cn
    mind = jnp.min(d2, axis=0, keepdims=True)
    iota0 = iota_s[...]
    zf = jnp.min(jnp.where(d2 == mind, iota0, jnp.float32(_S)),
                 axis=0, keepdims=True)
    onehot = (iota0 == zf).astype(jnp.float32)
    cz = lax.dot_general(c, onehot, (((0,), (0,)), ((), ())),
                         preferred_element_type=jnp.float32)
    for r in range(_R):
        vq_ref[0, 0, r] = cz[:, r * _C:(r + 1) * _C]
    z_ref[0, 0] = zf.astype(jnp.int32).reshape(_R, _C)
    e = jnp.maximum(mind, 0.0)
    e_ref[0, 0] = e.reshape(_R, _C)
    prev = jnp.where((h == 0) & (b == 0), 0.0, commit_ref[0, 0])
    commit_ref[0, 0] = prev + jnp.sum(e)


def kernel(vecs, c_sum, c_count):
    vt = jnp.swapaxes(vecs, 3, 4)
    ccnt = c_count.reshape(_H, _S, 1)

    def im_v(h, b):
        return (b, h, 0, 0, 0)

    def im_cb(h, b):
        return (h, 0, 0)

    def im_ze(h, b):
        return (b, h, 0, 0)

    vq, z_out, e_out, commit = pl.pallas_call(
        _vq_body,
        grid=(_H, _B),
        in_specs=[
            pl.BlockSpec((1, 1, _R, _K, _C), im_v),
            pl.BlockSpec((1, _S, _K), im_cb),
            pl.BlockSpec((1, _S, 1), im_cb),
        ],
        out_specs=[
            pl.BlockSpec((1, 1, _R, _K, _C), im_v),
            pl.BlockSpec((1, 1, _R, _C), im_ze),
            pl.BlockSpec((1, 1, _R, _C), im_ze),
            pl.BlockSpec((1, 1), lambda h, b: (0, 0), memory_space=pltpu.SMEM),
        ],
        out_shape=[
            jax.ShapeDtypeStruct((_B, _H, _R, _K, _C), jnp.float32),
            jax.ShapeDtypeStruct((_B, _H, _R, _C), jnp.int32),
            jax.ShapeDtypeStruct((_B, _H, _R, _C), jnp.float32),
            jax.ShapeDtypeStruct((1, 1), jnp.float32),
        ],
        scratch_shapes=[
            pltpu.VMEM((_S, _K), jnp.float32),
            pltpu.VMEM((_S, _K), jnp.float32),
            pltpu.VMEM((_S, 1), jnp.float32),
            pltpu.VMEM((_S, _N), jnp.float32),
        ],
    )(vt, c_sum, ccnt)

    vecs_hat = jnp.swapaxes(vq, 3, 4)
    l_commit = commit[0, 0] / jnp.float32(_B * _R * _C)
    l_codebook = jnp.zeros((), jnp.float32)
    return (vecs_hat, z_out, l_commit, l_codebook, e_out)
